# baseline (speedup 1.0000x reference)
import jax
import jax.numpy as jnp
from jax import lax
from jax.experimental import pallas as pl
from jax.experimental.pallas import tpu as pltpu

_NPS = 1024
_N = 4096
_B = 2
_BM = 512


def _dist_kernel(f_ref, ft_ref, o_ref):
    o_ref[...] = lax.dot_general(
        f_ref[...], ft_ref[...], (((1,), (0,)), ((), ())),
        preferred_element_type=jnp.float32)


def _fps_kernel(dist0_ref, F_ref, FT_ref, asq_ref, asqc_ref, P_ref, PT_ref,
                out_ref, row_s, sem):
    pos = (lax.broadcasted_iota(jnp.int32, (8, 128), 0) * 128
           + lax.broadcasted_iota(jnp.int32, (8, 128), 1))

    def argmax_flat(md):
        return jnp.argmax(md, axis=1)[0].astype(jnp.int32)

    def argmax_first(md):
        return jnp.argmax(md.reshape(1, _N), axis=1)[0].astype(jnp.int32)

    def row_copy(l):
        return pltpu.make_async_copy(
            dist0_ref.at[pl.ds(l, 1), :], row_s, sem)

    def combine(b, md, l, mv):
        a_l = asqc_ref[b, pl.ds(l, 1), :][0, 0]
        b_row = asq_ref[b:b + 1, :]
        row = (-2.0 * mv + a_l) + b_row
        return jnp.minimum(md, row)

    def fold_dma(md, l):
        return combine(0, md, l, row_s[...])

    def fold_mv(md, l):
        fr = F_ref[1, pl.ds(l, 1), :]
        mv = lax.dot_general(
            fr, FT_ref[0], (((1,), (0,)), ((), ())),
            preferred_element_type=jnp.float32)
        return combine(1, md, l, mv)

    init_md = jnp.full((8, 512), 1e10, dtype=jnp.float32)
    init_mf = jnp.full((1, _N), 1e10, dtype=jnp.float32)
    zeros_acc = jnp.zeros((8, 128), jnp.int32)

    row_copy(0).start()
    mf1 = fold_mv(init_mf, 0)
    row_copy(0).wait()
    mf0 = fold_dma(init_mf, 0)

    carry0 = (mf0, mf1, init_md, init_md,
              jnp.int32(0), jnp.int32(0),
              zeros_acc, zeros_acc, zeros_acc, zeros_acc)

    def body(t, c):
        mf = [c[0], c[1]]
        mdp = [c[2], c[3]]
        ldp = [c[4], c[5]]
        af = [c[6], c[7]]
        adp = [c[8], c[9]]
        nf = [argmax_flat(mf[b]) for b in range(_B)]
        row_copy(nf[0]).start()
        for b in range(_B):
            af[b] = jnp.where(pos == t, nf[b], af[b])
        for b in range(_B):
            l = ldp[b]
            px = PT_ref[b, 0]
            py = PT_ref[b, 1]
            pz = PT_ref[b, 2]
            cen = P_ref[b, pl.ds(l, 1), :]
            c0 = cen[0, 0]
            c1 = cen[0, 1]
            c2 = cen[0, 2]
            dx = px - c0
            dy = py - c1
            dz = pz - c2
            row = (dx * dx + dy * dy) + dz * dz
            md = jnp.minimum(mdp[b], row)
            nd = argmax_first(md)
            mdp[b] = md
            ldp[b] = nd
            adp[b] = jnp.where(pos == t, nd, adp[b])
        mf[1] = fold_mv(mf[1], nf[1])
        row_copy(nf[0]).wait()
        mf[0] = fold_dma(mf[0], nf[0])
        return (mf[0], mf[1], mdp[0], mdp[1],
                ldp[0], ldp[1],
                af[0], af[1], adp[0], adp[1])

    cN = lax.fori_loop(1, _NPS, body, carry0)
    for b in range(_B):
        out_ref[0, b] = cN[6 + b]
        out_ref[1, b] = cN[8 + b]


def kernel(points, features, npoint):
    F = jnp.concatenate([points, jnp.swapaxes(features, 1, 2)], axis=2)
    asq = jnp.sum(F ** 2, axis=-1)
    FT = jnp.swapaxes(F, 1, 2)
    PT8 = jnp.swapaxes(points, 1, 2).reshape(2, 3, 8, 512)

    dist0 = pl.pallas_call(
        _dist_kernel,
        grid=(_N // _BM,),
        in_specs=[
            pl.BlockSpec((_BM, 131), lambda i: (i, 0)),
            pl.BlockSpec((131, _N), lambda i: (0, 0)),
        ],
        out_specs=pl.BlockSpec((_BM, _N), lambda i: (i, 0)),
        out_shape=jax.ShapeDtypeStruct((_N, _N), jnp.float32),
    )(F[0], FT[0])

    out = pl.pallas_call(
        _fps_kernel,
        in_specs=[
            pl.BlockSpec(memory_space=pl.ANY),
            pl.BlockSpec(memory_space=pltpu.MemorySpace.VMEM),
            pl.BlockSpec(memory_space=pltpu.MemorySpace.VMEM),
            pl.BlockSpec(memory_space=pltpu.MemorySpace.VMEM),
            pl.BlockSpec(memory_space=pltpu.MemorySpace.VMEM),
            pl.BlockSpec(memory_space=pltpu.MemorySpace.VMEM),
            pl.BlockSpec(memory_space=pltpu.MemorySpace.VMEM),
        ],
        scratch_shapes=[
            pltpu.VMEM((1, _N), jnp.float32),
            pltpu.SemaphoreType.DMA,
        ],
        out_shape=jax.ShapeDtypeStruct((2, _B, 8, 128), jnp.int32),
    )(dist0, F, FT[1:2], asq, asq[..., None], points, PT8)

    idx = out.reshape(2, _B, _NPS)
    fps_idx = jnp.concatenate([idx[0], idx[1]], axis=1)
    return fps_idx + (jnp.asarray(npoint, dtype=jnp.int32) - _NPS)
